# baseline (speedup 1.0000x reference)
import functools

import jax
import jax.numpy as jnp
from jax import lax
from jax.experimental import pallas as pl
from jax.experimental.pallas import tpu as pltpu
from jax.experimental.pallas import tpu_sc as plsc

_INFO = plsc.get_sparse_core_info()
_NC = _INFO.num_cores
_NS = _INFO.num_subcores
_L = _INFO.num_lanes
_NW = _NC * _NS


@functools.lru_cache(maxsize=None)
def _make_sc_pool(B, S, E):
    rows_w = B // _NW
    nidx_w = rows_w * S
    ech = E // _L

    splits = []
    off = 0
    while off < S:
        n = min(128, S - off)
        splits.append((off, n))
        off += n

    mesh = plsc.VectorSubcoreMesh(core_axis_name="c", subcore_axis_name="s")

    @functools.partial(
        pl.kernel,
        out_type=jax.ShapeDtypeStruct((B, E), jnp.float32),
        mesh=mesh,
        scratch_types=[
            pltpu.VMEM((nidx_w,), jnp.int32),
            pltpu.VMEM((S, E), jnp.float32),
            pltpu.VMEM((rows_w, E), jnp.float32),
            pltpu.SemaphoreType.DMA,
        ],
        compiler_params=pltpu.CompilerParams(use_tc_tiling_on_sc=False),
    )
    def sc_pool(idx_hbm, table_hbm, out_hbm, idx_v, rows_v, sums_v, sem):
        wid = lax.axis_index("s") * _NC + lax.axis_index("c")
        base = wid * nidx_w
        pltpu.sync_copy(idx_hbm.at[pl.ds(base, nidx_w)], idx_v)

        def row_body(r, carry):
            roff = r * S
            cps = [
                pltpu.async_copy(
                    table_hbm.at[idx_v.at[pl.ds(roff + o, n)]],
                    rows_v.at[pl.ds(o, n)],
                    sem,
                )
                for (o, n) in splits
            ]
            for cp in cps:
                cp.wait()

            zero = jnp.zeros((_L,), jnp.float32)

            def acc_body(s, accs):
                return tuple(
                    accs[d] + rows_v[s, pl.ds(d * _L, _L)] for d in range(ech)
                )

            accs = lax.fori_loop(0, S, acc_body, (zero,) * ech)
            for d in range(ech):
                sums_v[r, pl.ds(d * _L, _L)] = accs[d]
            return carry

        lax.fori_loop(0, rows_w, row_body, 0)
        pltpu.sync_copy(sums_v, out_hbm.at[pl.ds(wid * rows_w, rows_w)])

    return sc_pool


def _head_body(s_ref, w_ref, b_ref, o_ref, *, inv_s):
    acc = jnp.dot(s_ref[...], w_ref[...], preferred_element_type=jnp.float32)
    o_ref[...] = acc * inv_s + b_ref[...]


def kernel(x, embed_table, fc_w, fc_b):
    B = x.shape[1]
    S = x.shape[3]
    E = embed_table.shape[1]
    C = fc_w.shape[0]

    idx = x.reshape(B * S).astype(jnp.int32)
    sums = _make_sc_pool(B, S, E)(idx, embed_table)

    head = pl.pallas_call(
        functools.partial(_head_body, inv_s=1.0 / S),
        out_shape=jax.ShapeDtypeStruct((B, C), jnp.float32),
    )
    w_t = jnp.transpose(fc_w).astype(jnp.float32)
    return head(sums, w_t, fc_b.reshape(1, C).astype(jnp.float32))


# speedup vs baseline: 1.1114x; 1.1114x over previous
import functools

import jax
import jax.numpy as jnp
from jax import lax
from jax.experimental import pallas as pl
from jax.experimental.pallas import tpu as pltpu
from jax.experimental.pallas import tpu_sc as plsc

_INFO = plsc.get_sparse_core_info()
_NC = _INFO.num_cores
_NS = _INFO.num_subcores
_L = _INFO.num_lanes
_NW = _NC * _NS


@functools.lru_cache(maxsize=None)
def _make_sc_pool(B, S, E):
    rows_w = B // _NW
    nidx_w = rows_w * S
    ech = E // _L

    splits = []
    off = 0
    while off < S:
        n = min(128, S - off)
        splits.append((off, n))
        off += n

    mesh = plsc.VectorSubcoreMesh(core_axis_name="c", subcore_axis_name="s")

    assert S % 8 == 0 and rows_w % 2 == 0
    unroll = 8
    assert S % unroll == 0

    @functools.partial(
        pl.kernel,
        out_type=jax.ShapeDtypeStruct((B, E), jnp.float32),
        mesh=mesh,
        scratch_types=[
            pltpu.VMEM((nidx_w,), jnp.int32),
            pltpu.VMEM((2, S, E), jnp.float32),
            pltpu.VMEM((rows_w, E), jnp.float32),
            pltpu.SemaphoreType.DMA,
            pltpu.SemaphoreType.DMA,
        ],
        compiler_params=pltpu.CompilerParams(use_tc_tiling_on_sc=False),
    )
    def sc_pool(idx_hbm, table_hbm, out_hbm, idx_v, rows_v, sums_v, sem0, sem1):
        wid = lax.axis_index("s") * _NC + lax.axis_index("c")
        base = wid * nidx_w
        pltpu.sync_copy(idx_hbm.at[pl.ds(base, nidx_w)], idx_v)

        def issue_row(r, buf, sem):
            roff = r * S
            for (o, n) in splits:
                pltpu.async_copy(
                    table_hbm.at[idx_v.at[pl.ds(roff + o, n)]],
                    buf.at[pl.ds(o, n)],
                    sem,
                )

        def wait_row(buf, sem):
            for (o, n) in splits:
                pltpu.make_async_copy(
                    table_hbm.at[idx_v.at[pl.ds(o, n)]],
                    buf.at[pl.ds(o, n)],
                    sem,
                ).wait()

        def accumulate(buf, r):
            zero = jnp.zeros((_L,), jnp.float32)

            def acc_body(i, accs):
                s0 = i * unroll
                accs = list(accs)
                for j in range(unroll):
                    for d in range(ech):
                        accs[d] = accs[d] + buf[s0 + j, pl.ds(d * _L, _L)]
                return tuple(accs)

            accs = lax.fori_loop(0, S // unroll, acc_body, (zero,) * ech)
            for d in range(ech):
                sums_v[r, pl.ds(d * _L, _L)] = accs[d]

        issue_row(0, rows_v.at[0], sem0)

        def pair_body(g, carry):
            r0 = g * 2
            wait_row(rows_v.at[0], sem0)
            issue_row(r0 + 1, rows_v.at[1], sem1)
            accumulate(rows_v.at[0], r0)
            wait_row(rows_v.at[1], sem1)

            @pl.when(r0 + 2 < rows_w)
            def _():
                issue_row(r0 + 2, rows_v.at[0], sem0)

            accumulate(rows_v.at[1], r0 + 1)
            return carry

        lax.fori_loop(0, rows_w // 2, pair_body, 0)
        pltpu.sync_copy(sums_v, out_hbm.at[pl.ds(wid * rows_w, rows_w)])

    return sc_pool


def _head_body(s_ref, w_ref, b_ref, o_ref, *, inv_s):
    acc = jnp.dot(s_ref[...], w_ref[...], preferred_element_type=jnp.float32)
    o_ref[...] = acc * inv_s + b_ref[...]


def kernel(x, embed_table, fc_w, fc_b):
    B = x.shape[1]
    S = x.shape[3]
    E = embed_table.shape[1]
    C = fc_w.shape[0]

    idx = x.reshape(B * S).astype(jnp.int32)
    sums = _make_sc_pool(B, S, E)(idx, embed_table)

    head = pl.pallas_call(
        functools.partial(_head_body, inv_s=1.0 / S),
        out_shape=jax.ShapeDtypeStruct((B, C), jnp.float32),
    )
    w_t = jnp.transpose(fc_w).astype(jnp.float32)
    return head(sums, w_t, fc_b.reshape(1, C).astype(jnp.float32))


# speedup vs baseline: 1.1117x; 1.0003x over previous
import functools

import jax
import jax.numpy as jnp
from jax import lax
from jax.experimental import pallas as pl
from jax.experimental.pallas import tpu as pltpu
from jax.experimental.pallas import tpu_sc as plsc

_INFO = plsc.get_sparse_core_info()
_NC = _INFO.num_cores
_NS = _INFO.num_subcores
_L = _INFO.num_lanes
_NW = _NC * _NS


@functools.lru_cache(maxsize=None)
def _make_sc_pool(B, S, E):
    rows_w = B // _NW
    nidx_w = rows_w * S
    ech = E // _L

    splits = []
    off = 0
    while off < S:
        n = min(128, S - off)
        splits.append((off, n))
        off += n

    mesh = plsc.VectorSubcoreMesh(core_axis_name="c", subcore_axis_name="s")

    assert S % 8 == 0 and rows_w % 2 == 0
    unroll = 8
    assert S % unroll == 0

    @functools.partial(
        pl.kernel,
        out_type=jax.ShapeDtypeStruct((B, E), jnp.float32),
        mesh=mesh,
        scratch_types=[
            pltpu.VMEM((nidx_w,), jnp.int32),
            pltpu.VMEM((2, S, E), jnp.float32),
            pltpu.VMEM((rows_w, E), jnp.float32),
            pltpu.SemaphoreType.DMA,
            pltpu.SemaphoreType.DMA,
        ],
        compiler_params=pltpu.CompilerParams(use_tc_tiling_on_sc=False),
    )
    def sc_pool(idx_hbm, table_hbm, out_hbm, idx_v, rows_v, sums_v, sem0, sem1):
        wid = lax.axis_index("s") * _NC + lax.axis_index("c")
        base = wid * nidx_w
        pltpu.sync_copy(idx_hbm.at[pl.ds(base, nidx_w)], idx_v)

        def issue_row(r, buf, sem):
            roff = r * S
            for (o, n) in splits:
                pltpu.async_copy(
                    table_hbm.at[idx_v.at[pl.ds(roff + o, n)]],
                    buf.at[pl.ds(o, n)],
                    sem,
                )

        def wait_row(buf, sem):
            for (o, n) in splits:
                pltpu.make_async_copy(
                    table_hbm.at[idx_v.at[pl.ds(o, n)]],
                    buf.at[pl.ds(o, n)],
                    sem,
                ).wait()

        def accumulate(buf, r):
            zero = jnp.zeros((_L,), jnp.float32)

            def acc_body(i, accs):
                s0 = i * unroll
                accs = list(accs)
                for j in range(unroll):
                    for d in range(ech):
                        accs[d] = accs[d] + buf[s0 + j, pl.ds(d * _L, _L)]
                return tuple(accs)

            accs = lax.fori_loop(0, S // unroll, acc_body, (zero,) * ech)
            for d in range(ech):
                sums_v[r, pl.ds(d * _L, _L)] = accs[d]

        issue_row(0, rows_v.at[0], sem0)

        def pair_body(g, carry):
            r0 = g * 2
            wait_row(rows_v.at[0], sem0)
            issue_row(r0 + 1, rows_v.at[1], sem1)
            accumulate(rows_v.at[0], r0)
            wait_row(rows_v.at[1], sem1)

            @pl.when(r0 + 2 < rows_w)
            def _():
                issue_row(r0 + 2, rows_v.at[0], sem0)

            accumulate(rows_v.at[1], r0 + 1)
            return carry

        lax.fori_loop(0, rows_w // 2, pair_body, 0)
        pltpu.sync_copy(sums_v, out_hbm.at[pl.ds(wid * rows_w, rows_w)])

    return sc_pool


def _head_body(s_ref, w_ref, b_ref, o_ref, *, inv_s):
    acc = jnp.dot(s_ref[...], w_ref[...], preferred_element_type=jnp.float32)
    o_ref[...] = acc * inv_s + b_ref[...]


def kernel(x, embed_table, fc_w, fc_b):
    B = x.shape[1]
    S = x.shape[3]
    E = embed_table.shape[1]
    C = fc_w.shape[0]

    idx = jax.lax.optimization_barrier(
        x.reshape(B * S // 128, 128).astype(jnp.int32)
    ).reshape(B * S)
    sums = _make_sc_pool(B, S, E)(idx, embed_table)

    head = pl.pallas_call(
        functools.partial(_head_body, inv_s=1.0 / S),
        out_shape=jax.ShapeDtypeStruct((B, C), jnp.float32),
    )
    w_t = jnp.transpose(fc_w).astype(jnp.float32)
    return head(sums, w_t, fc_b.reshape(1, C).astype(jnp.float32))
